# Initial kernel scaffold; baseline (speedup 1.0000x reference)
import functools

import jax
import jax.numpy as jnp
from jax import lax
from jax.experimental import pallas as pl
from jax.experimental.pallas import tpu as pltpu
from jax.experimental.pallas import tpu_sc as plsc

N = 10000
E = 160000
IN_CH = 7
EMBED = 128

NC = 2
NS = 16
NW = NC * NS
K = 128
CH = 40
EPW = CH * K
EP = NW * EPW
NA = 10240
RPT = NA // NS

_MESH = dict(core_axis_name="c", subcore_axis_name="s", num_cores=NC,
             num_subcores=NS)


def _gather_body(xpad_hbm, srcp_hbm, xj_hbm, idx_v, rows_v, sem):
    c = lax.axis_index("c")
    s = lax.axis_index("s")
    wid = s * NC + c
    pltpu.sync_copy(srcp_hbm.at[wid], idx_v)

    def body(j, carry):
        pltpu.async_copy(xpad_hbm.at[idx_v.at[j]],
                         rows_v.at[pl.ds(j * K, K)], sem).wait()
        return carry

    lax.fori_loop(0, CH, body, 0)
    pltpu.sync_copy(rows_v, xj_hbm.at[pl.ds(wid * EPW, EPW)])


def _sc_gather(xpad, srcp):
    fn = functools.partial(
        pl.kernel,
        out_type=jax.ShapeDtypeStruct((EP, 16), jnp.float32),
        mesh=plsc.VectorSubcoreMesh(**_MESH),
        scratch_types=[
            pltpu.VMEM((CH, K), jnp.int32),
            pltpu.VMEM((EPW, 16), jnp.float32),
            pltpu.SemaphoreType.DMA,
        ],
    )(_gather_body)
    return fn(xpad, srcp)


def _scatter_body(msg_hbm, dstp_hbm, acc_hbm, cnt_hbm,
                  idx_v, stage_v, ones_v, zc_v, acc_s, cnt_s):
    c = lax.axis_index("c")
    s = lax.axis_index("s")
    wid = s * NC + c

    zf = jnp.zeros((16,), jnp.float32)
    of = jnp.ones((16,), jnp.float32)

    def zstage(r, carry):
        for cc in range(8):
            stage_v[r, pl.ds(cc * 16, 16)] = zf
        return carry

    lax.fori_loop(0, K, zstage, 0)

    def zsmall(r, carry):
        ones_v[r, :] = of
        zc_v[r, :] = zf
        return carry

    lax.fori_loop(0, K, zsmall, 0)

    for t in range(RPT // K):
        pltpu.sync_copy(stage_v, acc_s.at[pl.ds(s * RPT + t * K, K)])
        pltpu.sync_copy(zc_v, cnt_s.at[pl.ds(s * RPT + t * K, K)])
    plsc.subcore_barrier()

    pltpu.sync_copy(dstp_hbm.at[wid], idx_v)

    def body(j, carry):
        pltpu.sync_copy(msg_hbm.at[pl.ds(wid * EPW + j * K, K)], stage_v)
        pltpu.sync_copy(stage_v, acc_s.at[idx_v.at[j]], add=True)
        pltpu.sync_copy(ones_v, cnt_s.at[idx_v.at[j]], add=True)
        return carry

    lax.fori_loop(0, CH, body, 0)
    plsc.subcore_barrier()

    pltpu.sync_copy(acc_s.at[pl.ds(s * RPT, RPT)],
                    acc_hbm.at[c, pl.ds(s * RPT, RPT)])
    pltpu.sync_copy(cnt_s.at[pl.ds(s * RPT, RPT)],
                    cnt_hbm.at[c, pl.ds(s * RPT, RPT)])


def _sc_scatter(msg, dstp):
    fn = functools.partial(
        pl.kernel,
        out_type=(jax.ShapeDtypeStruct((NC, NA, EMBED), jnp.float32),
                  jax.ShapeDtypeStruct((NC, NA, 16), jnp.float32)),
        mesh=plsc.VectorSubcoreMesh(**_MESH),
        scratch_types=[
            pltpu.VMEM((CH, K), jnp.int32),
            pltpu.VMEM((K, EMBED), jnp.float32),
            pltpu.VMEM((K, 16), jnp.float32),
            pltpu.VMEM((K, 16), jnp.float32),
            pltpu.VMEM_SHARED((NA, EMBED), jnp.float32),
            pltpu.VMEM_SHARED((NA, 16), jnp.float32),
        ],
    )(_scatter_body)
    return fn(msg, dstp)


def _msg_kernel(ea_ref, xj_ref, w1_ref, b1_ref, w2_ref, b2_ref, out_ref):
    h = jnp.dot(ea_ref[...], w1_ref[...],
                preferred_element_type=jnp.float32) + b1_ref[...]
    h = 0.5 * h * (1.0 + lax.erf(h * 0.7071067811865476))
    xj = xj_ref[...]
    acc = None
    for i in range(IN_CH):
        gi = jnp.dot(h, w2_ref[i],
                     preferred_element_type=jnp.float32) + b2_ref[i:i + 1, :]
        term = xj[:, i:i + 1] * gi
        acc = term if acc is None else acc + term
    out_ref[...] = acc


def _tc_message(eap, xj, W1p, b1r, W2r, b2p):
    TE = 2048
    return pl.pallas_call(
        _msg_kernel,
        grid=(EP // TE,),
        in_specs=[
            pl.BlockSpec((TE, 8), lambda i: (i, 0)),
            pl.BlockSpec((TE, 16), lambda i: (i, 0)),
            pl.BlockSpec((8, EMBED), lambda i: (0, 0)),
            pl.BlockSpec((1, EMBED), lambda i: (0, 0)),
            pl.BlockSpec((IN_CH, EMBED, EMBED), lambda i: (0, 0, 0)),
            pl.BlockSpec((8, EMBED), lambda i: (0, 0)),
        ],
        out_specs=pl.BlockSpec((TE, EMBED), lambda i: (i, 0)),
        out_shape=jax.ShapeDtypeStruct((EP, EMBED), jnp.float32),
    )(eap, xj, W1p, b1r, W2r, b2p)


def _final_kernel(acc_ref, cnt_ref, x_ref, root_ref, bias_ref, out_ref):
    summed = acc_ref[0] + acc_ref[1]
    cvec = cnt_ref[0, :, 0:1] + cnt_ref[1, :, 0:1]
    cvec = jnp.maximum(cvec, 1.0)
    xr = jnp.dot(x_ref[...], root_ref[...], preferred_element_type=jnp.float32)
    out_ref[...] = summed / cvec + xr + bias_ref[...]


def _tc_final(accs, cnts, xpad, rootp, biasr):
    TN = 2000
    return pl.pallas_call(
        _final_kernel,
        grid=(N // TN,),
        in_specs=[
            pl.BlockSpec((NC, TN, EMBED), lambda i: (0, i, 0)),
            pl.BlockSpec((NC, TN, 16), lambda i: (0, i, 0)),
            pl.BlockSpec((TN, 16), lambda i: (i, 0)),
            pl.BlockSpec((16, EMBED), lambda i: (0, 0)),
            pl.BlockSpec((1, EMBED), lambda i: (0, 0)),
        ],
        out_specs=pl.BlockSpec((TN, EMBED), lambda i: (i, 0)),
        out_shape=jax.ShapeDtypeStruct((N, EMBED), jnp.float32),
    )(accs, cnts, xpad, rootp, biasr)


def kernel(x, edge_index, edge_attr, W1, b1, W2, b2, root, bias):
    src = edge_index[0].astype(jnp.int32)
    dst = edge_index[1].astype(jnp.int32)
    pad = EP - E
    srcp = jnp.concatenate([src, jnp.zeros((pad,), jnp.int32)]).reshape(NW, CH, K)
    dstp = jnp.concatenate([dst, jnp.full((pad,), N, jnp.int32)]).reshape(NW, CH, K)
    eap = jnp.pad(edge_attr, ((0, pad), (0, 2)))
    xpad = jnp.pad(x, ((0, 0), (0, 16 - IN_CH)))
    W1p = jnp.pad(W1, ((0, 2), (0, 0)))
    b1r = b1.reshape(1, EMBED)
    W2r = W2.reshape(EMBED, IN_CH, EMBED).transpose(1, 0, 2)
    b2p = jnp.pad(b2.reshape(IN_CH, EMBED), ((0, 1), (0, 0)))
    rootp = jnp.pad(root, ((0, 16 - IN_CH), (0, 0)))
    biasr = bias.reshape(1, EMBED)

    xj = _sc_gather(xpad, srcp)
    msg = _tc_message(eap, xj, W1p, b1r, W2r, b2p)
    accs, cnts = _sc_scatter(msg, dstp)
    return _tc_final(accs, cnts, xpad, rootp, biasr)


# speedup vs baseline: 3.6783x; 3.6783x over previous
import functools

import jax
import jax.numpy as jnp
from jax import lax
from jax.experimental import pallas as pl
from jax.experimental.pallas import tpu as pltpu
from jax.experimental.pallas import tpu_sc as plsc

N = 10000
E = 160000
IN_CH = 7
EMBED = 128

NC = 2
NS = 16
NW = NC * NS
K = 128
CH = 40
EPW = CH * K
EP = NW * EPW
NA = 10240
RPT = NA // NS

_MESH = dict(core_axis_name="c", subcore_axis_name="s", num_cores=NC,
             num_subcores=NS)


def _gather_body(xpad_hbm, srcp_hbm, xj_hbm, idx_v, rows_v, sem):
    c = lax.axis_index("c")
    s = lax.axis_index("s")
    wid = s * NC + c
    pltpu.sync_copy(srcp_hbm.at[wid], idx_v)

    def body(j, carry):
        pltpu.async_copy(xpad_hbm.at[idx_v.at[j]],
                         rows_v.at[pl.ds(j * K, K)], sem).wait()
        return carry

    lax.fori_loop(0, CH, body, 0)
    pltpu.sync_copy(rows_v, xj_hbm.at[pl.ds(wid * EPW, EPW)])


def _sc_gather(xpad, srcp):
    fn = functools.partial(
        pl.kernel,
        out_type=jax.ShapeDtypeStruct((EP, 16), jnp.float32),
        mesh=plsc.VectorSubcoreMesh(**_MESH),
        scratch_types=[
            pltpu.VMEM((CH, K), jnp.int32),
            pltpu.VMEM((EPW, 16), jnp.float32),
            pltpu.SemaphoreType.DMA,
        ],
        compiler_params=pltpu.CompilerParams(use_tc_tiling_on_sc=False),
    )(_gather_body)
    return fn(xpad, srcp)


def _scatter_body(msg_hbm, dstp_hbm, acc_hbm, cnt_hbm,
                  idx_v, stage_v, ones_v, zc_v, acc_s, cnt_s):
    c = lax.axis_index("c")
    s = lax.axis_index("s")
    wid = s * NC + c

    zf = jnp.zeros((16,), jnp.float32)
    of = jnp.ones((16,), jnp.float32)

    def zstage(r, carry):
        for cc in range(8):
            stage_v[r, pl.ds(cc * 16, 16)] = zf
        return carry

    lax.fori_loop(0, K, zstage, 0)

    def zsmall(r, carry):
        ones_v[r, :] = of
        zc_v[r, :] = zf
        return carry

    lax.fori_loop(0, K, zsmall, 0)

    for t in range(RPT // K):
        pltpu.sync_copy(stage_v, acc_s.at[pl.ds(s * RPT + t * K, K)])
        pltpu.sync_copy(zc_v, cnt_s.at[pl.ds(s * RPT + t * K, K)])
    plsc.subcore_barrier()

    pltpu.sync_copy(dstp_hbm.at[wid], idx_v)

    def body(j, carry):
        pltpu.sync_copy(msg_hbm.at[pl.ds(wid * EPW + j * K, K)], stage_v)
        pltpu.sync_copy(stage_v, acc_s.at[idx_v.at[j]], add=True)
        pltpu.sync_copy(ones_v, cnt_s.at[idx_v.at[j]], add=True)
        return carry

    lax.fori_loop(0, CH, body, 0)
    plsc.subcore_barrier()

    pltpu.sync_copy(acc_s.at[pl.ds(s * RPT, RPT)],
                    acc_hbm.at[c, pl.ds(s * RPT, RPT)])
    pltpu.sync_copy(cnt_s.at[pl.ds(s * RPT, RPT)],
                    cnt_hbm.at[c, pl.ds(s * RPT, RPT)])


def _sc_scatter(msg, dstp):
    fn = functools.partial(
        pl.kernel,
        out_type=(jax.ShapeDtypeStruct((NC, NA, EMBED), jnp.float32),
                  jax.ShapeDtypeStruct((NC, NA, 16), jnp.float32)),
        mesh=plsc.VectorSubcoreMesh(**_MESH),
        scratch_types=[
            pltpu.VMEM((CH, K), jnp.int32),
            pltpu.VMEM((K, EMBED), jnp.float32),
            pltpu.VMEM((K, 16), jnp.float32),
            pltpu.VMEM((K, 16), jnp.float32),
            pltpu.VMEM_SHARED((NA, EMBED), jnp.float32),
            pltpu.VMEM_SHARED((NA, 16), jnp.float32),
        ],
        compiler_params=pltpu.CompilerParams(use_tc_tiling_on_sc=False),
    )(_scatter_body)
    return fn(msg, dstp)


def _msg_kernel(ea_ref, xj_ref, w1_ref, b1_ref, w2_ref, b2_ref, out_ref):
    h = jnp.dot(ea_ref[...], w1_ref[...],
                preferred_element_type=jnp.float32) + b1_ref[...]
    h = 0.5 * h * (1.0 + lax.erf(h * 0.7071067811865476))
    xj = xj_ref[...]
    acc = None
    for i in range(IN_CH):
        gi = jnp.dot(h, w2_ref[i],
                     preferred_element_type=jnp.float32) + b2_ref[i:i + 1, :]
        term = xj[:, i:i + 1] * gi
        acc = term if acc is None else acc + term
    out_ref[...] = acc


def _tc_message(eap, xj, W1p, b1r, W2r, b2p):
    TE = 2048
    return pl.pallas_call(
        _msg_kernel,
        grid=(EP // TE,),
        in_specs=[
            pl.BlockSpec((TE, 8), lambda i: (i, 0)),
            pl.BlockSpec((TE, 16), lambda i: (i, 0)),
            pl.BlockSpec((8, EMBED), lambda i: (0, 0)),
            pl.BlockSpec((1, EMBED), lambda i: (0, 0)),
            pl.BlockSpec((IN_CH, EMBED, EMBED), lambda i: (0, 0, 0)),
            pl.BlockSpec((8, EMBED), lambda i: (0, 0)),
        ],
        out_specs=pl.BlockSpec((TE, EMBED), lambda i: (i, 0)),
        out_shape=jax.ShapeDtypeStruct((EP, EMBED), jnp.float32),
    )(eap, xj, W1p, b1r, W2r, b2p)


def _final_kernel(acc_ref, cnt_ref, x_ref, root_ref, bias_ref, out_ref):
    summed = acc_ref[0] + acc_ref[1]
    cvec = cnt_ref[0, :, 0:1] + cnt_ref[1, :, 0:1]
    cvec = jnp.maximum(cvec, 1.0)
    xr = jnp.dot(x_ref[...], root_ref[...], preferred_element_type=jnp.float32)
    out_ref[...] = summed / cvec + xr + bias_ref[...]


def _tc_final(accs, cnts, xpad, rootp, biasr):
    TN = 2000
    return pl.pallas_call(
        _final_kernel,
        grid=(N // TN,),
        in_specs=[
            pl.BlockSpec((NC, TN, EMBED), lambda i: (0, i, 0)),
            pl.BlockSpec((NC, TN, 16), lambda i: (0, i, 0)),
            pl.BlockSpec((TN, 16), lambda i: (i, 0)),
            pl.BlockSpec((16, EMBED), lambda i: (0, 0)),
            pl.BlockSpec((1, EMBED), lambda i: (0, 0)),
        ],
        out_specs=pl.BlockSpec((TN, EMBED), lambda i: (i, 0)),
        out_shape=jax.ShapeDtypeStruct((N, EMBED), jnp.float32),
    )(accs, cnts, xpad, rootp, biasr)


def kernel(x, edge_index, edge_attr, W1, b1, W2, b2, root, bias):
    src = edge_index[0].astype(jnp.int32)
    dst = edge_index[1].astype(jnp.int32)
    pad = EP - E
    srcp = jnp.concatenate([src, jnp.zeros((pad,), jnp.int32)]).reshape(NW, CH, K)
    dstp = jnp.concatenate([dst, jnp.full((pad,), N, jnp.int32)]).reshape(NW, CH, K)
    eap = jnp.pad(edge_attr, ((0, pad), (0, 2)))
    xpad = jnp.pad(x, ((0, 0), (0, 16 - IN_CH)))
    W1p = jnp.pad(W1, ((0, 2), (0, 0)))
    b1r = b1.reshape(1, EMBED)
    W2r = W2.reshape(EMBED, IN_CH, EMBED).transpose(1, 0, 2)
    b2p = jnp.pad(b2.reshape(IN_CH, EMBED), ((0, 1), (0, 0)))
    rootp = jnp.pad(root, ((0, 16 - IN_CH), (0, 0)))
    biasr = bias.reshape(1, EMBED)

    xj = _sc_gather(xpad, srcp)
    msg = _tc_message(eap, xj, W1p, b1r, W2r, b2p)
    accs, cnts = _sc_scatter(msg, dstp)
    return _tc_final(accs, cnts, xpad, rootp, biasr)


# speedup vs baseline: 4.2297x; 1.1499x over previous
import functools

import jax
import jax.numpy as jnp
from jax import lax
from jax.experimental import pallas as pl
from jax.experimental.pallas import tpu as pltpu
from jax.experimental.pallas import tpu_sc as plsc

N = 10000
E = 160000
IN_CH = 7
EMBED = 128

NC = 2
NS = 16
NW = NC * NS
K = 128
CH = 40
KS = 64
CH2 = 80
EPW = CH * K
EP = NW * EPW
NA = 10240
RPT = NA // NS
XPT = N // NS
NBUF = 4

_MESH = dict(core_axis_name="c", subcore_axis_name="s", num_cores=NC,
             num_subcores=NS)
_SC_PARAMS = pltpu.CompilerParams(use_tc_tiling_on_sc=False)


def _gather_body(xpad_hbm, srcp_hbm, dstp_hbm, xj_hbm, cnt_hbm,
                 idx_v, didx_v, rows_v, ones_v, zc_v, xs_s, cnt_s, sem):
    c = lax.axis_index("c")
    s = lax.axis_index("s")
    wid = s * NC + c

    pltpu.sync_copy(xpad_hbm.at[pl.ds(s * XPT, XPT)],
                    xs_s.at[pl.ds(s * XPT, XPT)])
    pltpu.sync_copy(srcp_hbm.at[wid], idx_v)
    pltpu.sync_copy(dstp_hbm.at[wid], didx_v)

    zf = jnp.zeros((16,), jnp.float32)
    of = jnp.ones((16,), jnp.float32)

    def fill(r, carry):
        ones_v[r, :] = of
        zc_v[r, :] = zf
        return carry

    lax.fori_loop(0, K, fill, 0)
    for t in range(RPT // K):
        pltpu.sync_copy(zc_v, cnt_s.at[pl.ds(s * RPT + t * K, K)])
    plsc.subcore_barrier()

    def fire(j, carry):
        pltpu.async_copy(xs_s.at[idx_v.at[j]],
                         rows_v.at[pl.ds(j * K, K)], sem)
        return carry

    lax.fori_loop(0, CH, fire, 0)

    def ones_scatter(j, carry):
        pltpu.sync_copy(ones_v, cnt_s.at[didx_v.at[j]], add=True)
        return carry

    lax.fori_loop(0, CH, ones_scatter, 0)

    pltpu.make_async_copy(xj_hbm.at[pl.ds(wid * EPW, EPW)], rows_v,
                          sem).wait()
    pltpu.sync_copy(rows_v, xj_hbm.at[pl.ds(wid * EPW, EPW)])
    plsc.subcore_barrier()
    pltpu.sync_copy(cnt_s.at[pl.ds(s * XPT, XPT)],
                    cnt_hbm.at[c, pl.ds(s * XPT, XPT)])


def _sc_gather(xpad, srcp, dstp):
    fn = functools.partial(
        pl.kernel,
        out_type=(jax.ShapeDtypeStruct((EP, 16), jnp.float32),
                  jax.ShapeDtypeStruct((NC, N, 16), jnp.float32)),
        mesh=plsc.VectorSubcoreMesh(**_MESH),
        scratch_types=[
            pltpu.VMEM((CH, K), jnp.int32),
            pltpu.VMEM((CH, K), jnp.int32),
            pltpu.VMEM((EPW, 16), jnp.float32),
            pltpu.VMEM((K, 16), jnp.float32),
            pltpu.VMEM((K, 16), jnp.float32),
            pltpu.VMEM_SHARED((N, 16), jnp.float32),
            pltpu.VMEM_SHARED((NA, 16), jnp.float32),
            pltpu.SemaphoreType.DMA,
        ],
        compiler_params=_SC_PARAMS,
    )(_gather_body)
    return fn(xpad, srcp, dstp)


def _scatter_body(msg_hbm, dstp_hbm, acc_hbm,
                  idx_v, st0, st1, st2, st3, acc_s, sm0, sm1, sm2, sm3):
    c = lax.axis_index("c")
    s = lax.axis_index("s")
    wid = s * NC + c
    base = wid * EPW
    stages = (st0, st1, st2, st3)
    sems = (sm0, sm1, sm2, sm3)

    zf = jnp.zeros((16,), jnp.float32)

    def zstage(r, carry):
        for cc in range(8):
            st0[r, pl.ds(cc * 16, 16)] = zf
        return carry

    lax.fori_loop(0, KS, zstage, 0)
    for t in range(RPT // KS):
        pltpu.sync_copy(st0, acc_s.at[pl.ds(s * RPT + t * KS, KS)])
    plsc.subcore_barrier()

    pltpu.sync_copy(dstp_hbm.at[wid], idx_v)

    for b in range(NBUF):
        pltpu.async_copy(msg_hbm.at[pl.ds(base + b * KS, KS)], stages[b],
                         sems[b])

    def outer(o, carry):
        for b in range(NBUF):
            j = o * NBUF + b
            pltpu.make_async_copy(msg_hbm.at[pl.ds(base + j * KS, KS)],
                                  stages[b], sems[b]).wait()
            pltpu.sync_copy(stages[b], acc_s.at[idx_v.at[j]], add=True)

            @pl.when(o < CH2 // NBUF - 1)
            def _fire(b=b, j=j):
                pltpu.async_copy(msg_hbm.at[pl.ds(base + (j + NBUF) * KS, KS)],
                                 stages[b], sems[b])

        return carry

    lax.fori_loop(0, CH2 // NBUF, outer, 0)
    plsc.subcore_barrier()

    pltpu.sync_copy(acc_s.at[pl.ds(s * XPT, XPT)],
                    acc_hbm.at[c, pl.ds(s * XPT, XPT)])


def _sc_scatter(msg, dstp_s):
    fn = functools.partial(
        pl.kernel,
        out_type=jax.ShapeDtypeStruct((NC, N, EMBED), jnp.float32),
        mesh=plsc.VectorSubcoreMesh(**_MESH),
        scratch_types=[
            pltpu.VMEM((CH2, KS), jnp.int32),
            pltpu.VMEM((KS, EMBED), jnp.float32),
            pltpu.VMEM((KS, EMBED), jnp.float32),
            pltpu.VMEM((KS, EMBED), jnp.float32),
            pltpu.VMEM((KS, EMBED), jnp.float32),
            pltpu.VMEM_SHARED((NA, EMBED), jnp.float32),
            pltpu.SemaphoreType.DMA,
            pltpu.SemaphoreType.DMA,
            pltpu.SemaphoreType.DMA,
            pltpu.SemaphoreType.DMA,
        ],
        compiler_params=_SC_PARAMS,
    )(_scatter_body)
    return fn(msg, dstp_s)


def _msg_kernel(ea_ref, xj_ref, w1_ref, b1_ref, w2_ref, b2_ref, out_ref):
    h = jnp.dot(ea_ref[...], w1_ref[...],
                preferred_element_type=jnp.float32) + b1_ref[...]
    h = 0.5 * h * (1.0 + lax.erf(h * 0.7071067811865476))
    xj = xj_ref[...]
    acc = None
    for i in range(IN_CH):
        gi = jnp.dot(h, w2_ref[i],
                     preferred_element_type=jnp.float32) + b2_ref[i:i + 1, :]
        term = xj[:, i:i + 1] * gi
        acc = term if acc is None else acc + term
    out_ref[...] = acc


def _tc_message(eap, xj, W1p, b1r, W2r, b2p):
    TE = 2048
    return pl.pallas_call(
        _msg_kernel,
        grid=(EP // TE,),
        in_specs=[
            pl.BlockSpec((TE, 8), lambda i: (i, 0)),
            pl.BlockSpec((TE, 16), lambda i: (i, 0)),
            pl.BlockSpec((8, EMBED), lambda i: (0, 0)),
            pl.BlockSpec((1, EMBED), lambda i: (0, 0)),
            pl.BlockSpec((IN_CH, EMBED, EMBED), lambda i: (0, 0, 0)),
            pl.BlockSpec((8, EMBED), lambda i: (0, 0)),
        ],
        out_specs=pl.BlockSpec((TE, EMBED), lambda i: (i, 0)),
        out_shape=jax.ShapeDtypeStruct((EP, EMBED), jnp.float32),
    )(eap, xj, W1p, b1r, W2r, b2p)


def _final_kernel(acc_ref, cnt_ref, x_ref, root_ref, bias_ref, out_ref):
    summed = acc_ref[0] + acc_ref[1]
    cvec = cnt_ref[0, :, 0:1] + cnt_ref[1, :, 0:1]
    cvec = jnp.maximum(cvec, 1.0)
    xr = jnp.dot(x_ref[...], root_ref[...], preferred_element_type=jnp.float32)
    out_ref[...] = summed / cvec + xr + bias_ref[...]


def _tc_final(accs, cnts, xpad, rootp, biasr):
    TN = 2000
    return pl.pallas_call(
        _final_kernel,
        grid=(N // TN,),
        in_specs=[
            pl.BlockSpec((NC, TN, EMBED), lambda i: (0, i, 0)),
            pl.BlockSpec((NC, TN, 16), lambda i: (0, i, 0)),
            pl.BlockSpec((TN, 16), lambda i: (i, 0)),
            pl.BlockSpec((16, EMBED), lambda i: (0, 0)),
            pl.BlockSpec((1, EMBED), lambda i: (0, 0)),
        ],
        out_specs=pl.BlockSpec((TN, EMBED), lambda i: (i, 0)),
        out_shape=jax.ShapeDtypeStruct((N, EMBED), jnp.float32),
    )(accs, cnts, xpad, rootp, biasr)


def kernel(x, edge_index, edge_attr, W1, b1, W2, b2, root, bias):
    src = edge_index[0].astype(jnp.int32)
    dst = edge_index[1].astype(jnp.int32)
    pad = EP - E
    srcp = jnp.concatenate([src, jnp.zeros((pad,), jnp.int32)]).reshape(NW, CH, K)
    dstp_full = jnp.concatenate([dst, jnp.full((pad,), N, jnp.int32)])
    dstp = dstp_full.reshape(NW, CH, K)
    dstp_s = dstp_full.reshape(NW, CH2, KS)
    eap = jnp.pad(edge_attr, ((0, pad), (0, 2)))
    xpad = jnp.pad(x, ((0, 0), (0, 16 - IN_CH)))
    W1p = jnp.pad(W1, ((0, 2), (0, 0)))
    b1r = b1.reshape(1, EMBED)
    W2r = W2.reshape(EMBED, IN_CH, EMBED).transpose(1, 0, 2)
    b2p = jnp.pad(b2.reshape(IN_CH, EMBED), ((0, 1), (0, 0)))
    rootp = jnp.pad(root, ((0, 16 - IN_CH), (0, 0)))
    biasr = bias.reshape(1, EMBED)

    xj, cnts = _sc_gather(xpad, srcp, dstp)
    msg = _tc_message(eap, xj, W1p, b1r, W2r, b2p)
    accs = _sc_scatter(msg, dstp_s)
    return _tc_final(accs, cnts, xpad, rootp, biasr)


# speedup vs baseline: 4.2713x; 1.0098x over previous
import functools

import jax
import jax.numpy as jnp
from jax import lax
from jax.experimental import pallas as pl
from jax.experimental.pallas import tpu as pltpu
from jax.experimental.pallas import tpu_sc as plsc

N = 10000
E = 160000
IN_CH = 7
EMBED = 128

NC = 2
NS = 16
NW = NC * NS
K = 128
CH = 40
KS = 64
CH2 = 80
EPW = CH * K
EP = NW * EPW
NA = 10240
RPT = NA // NS
XPT = N // NS
NBUF = 4

_MESH = dict(core_axis_name="c", subcore_axis_name="s", num_cores=NC,
             num_subcores=NS)
_SC_PARAMS = pltpu.CompilerParams(use_tc_tiling_on_sc=False)


def _gather_body(xpad_hbm, srcp_hbm, dstp_hbm, xj_hbm, cnt_hbm,
                 idx_v, didx_v, rows_v, ones_v, zc_v, xs_s, cnt_s, sem):
    c = lax.axis_index("c")
    s = lax.axis_index("s")
    wid = s * NC + c

    pltpu.sync_copy(xpad_hbm.at[pl.ds(s * XPT, XPT)],
                    xs_s.at[pl.ds(s * XPT, XPT)])
    pltpu.sync_copy(srcp_hbm.at[wid], idx_v)
    pltpu.sync_copy(dstp_hbm.at[wid], didx_v)

    zf = jnp.zeros((16,), jnp.float32)
    of = jnp.ones((16,), jnp.float32)

    def fill(r, carry):
        ones_v[r, :] = of
        zc_v[r, :] = zf
        return carry

    lax.fori_loop(0, K, fill, 0)
    for t in range(RPT // K):
        pltpu.sync_copy(zc_v, cnt_s.at[pl.ds(s * RPT + t * K, K)])
    plsc.subcore_barrier()

    def fire(j, carry):
        pltpu.async_copy(xs_s.at[idx_v.at[j]],
                         rows_v.at[pl.ds(j * K, K)], sem)
        return carry

    lax.fori_loop(0, CH, fire, 0)

    def ones_scatter(j, carry):
        pltpu.sync_copy(ones_v, cnt_s.at[didx_v.at[j]], add=True)
        return carry

    lax.fori_loop(0, CH, ones_scatter, 0)

    pltpu.make_async_copy(xj_hbm.at[pl.ds(wid * EPW, EPW)], rows_v,
                          sem).wait()
    pltpu.sync_copy(rows_v, xj_hbm.at[pl.ds(wid * EPW, EPW)])
    plsc.subcore_barrier()
    pltpu.sync_copy(cnt_s.at[pl.ds(s * XPT, XPT)],
                    cnt_hbm.at[c, pl.ds(s * XPT, XPT)])


def _sc_gather(xpad, srcp, dstp):
    fn = functools.partial(
        pl.kernel,
        out_type=(jax.ShapeDtypeStruct((EP, 16), jnp.float32),
                  jax.ShapeDtypeStruct((NC, N, 16), jnp.float32)),
        mesh=plsc.VectorSubcoreMesh(**_MESH),
        scratch_types=[
            pltpu.VMEM((CH, K), jnp.int32),
            pltpu.VMEM((CH, K), jnp.int32),
            pltpu.VMEM((EPW, 16), jnp.float32),
            pltpu.VMEM((K, 16), jnp.float32),
            pltpu.VMEM((K, 16), jnp.float32),
            pltpu.VMEM_SHARED((N, 16), jnp.float32),
            pltpu.VMEM_SHARED((NA, 16), jnp.float32),
            pltpu.SemaphoreType.DMA,
        ],
        compiler_params=_SC_PARAMS,
    )(_gather_body)
    return fn(xpad, srcp, dstp)


def _scatter_body(msg_hbm, dstp_hbm, acc_hbm,
                  idx_v, st0, st1, st2, st3, acc_s, sm0, sm1, sm2, sm3):
    c = lax.axis_index("c")
    s = lax.axis_index("s")
    wid = s * NC + c
    base = wid * EPW
    stages = (st0, st1, st2, st3)
    sems = (sm0, sm1, sm2, sm3)

    zf = jnp.zeros((16,), jnp.float32)

    def zstage(r, carry):
        for cc in range(8):
            st0[r, pl.ds(cc * 16, 16)] = zf
        return carry

    lax.fori_loop(0, KS, zstage, 0)
    for t in range(RPT // KS):
        pltpu.sync_copy(st0, acc_s.at[pl.ds(s * RPT + t * KS, KS)])
    plsc.subcore_barrier()

    pltpu.sync_copy(dstp_hbm.at[wid], idx_v)

    for b in range(NBUF):
        pltpu.async_copy(msg_hbm.at[pl.ds(base + b * KS, KS)], stages[b],
                         sems[b])

    def outer(o, carry):
        for b in range(NBUF):
            j = o * NBUF + b
            pltpu.make_async_copy(msg_hbm.at[pl.ds(base + j * KS, KS)],
                                  stages[b], sems[b]).wait()
            pltpu.sync_copy(stages[b], acc_s.at[idx_v.at[j]], add=True)

            @pl.when(o < CH2 // NBUF - 1)
            def _fire(b=b, j=j):
                pltpu.async_copy(msg_hbm.at[pl.ds(base + (j + NBUF) * KS, KS)],
                                 stages[b], sems[b])

        return carry

    lax.fori_loop(0, CH2 // NBUF, outer, 0)
    plsc.subcore_barrier()

    pltpu.sync_copy(acc_s.at[pl.ds(s * XPT, XPT)],
                    acc_hbm.at[c, pl.ds(s * XPT, XPT)])


def _sc_scatter(msg, dstp_s):
    fn = functools.partial(
        pl.kernel,
        out_type=jax.ShapeDtypeStruct((NC, N, EMBED), jnp.float32),
        mesh=plsc.VectorSubcoreMesh(**_MESH),
        scratch_types=[
            pltpu.VMEM((CH2, KS), jnp.int32),
            pltpu.VMEM((KS, EMBED), jnp.float32),
            pltpu.VMEM((KS, EMBED), jnp.float32),
            pltpu.VMEM((KS, EMBED), jnp.float32),
            pltpu.VMEM((KS, EMBED), jnp.float32),
            pltpu.VMEM_SHARED((NA, EMBED), jnp.float32),
            pltpu.SemaphoreType.DMA,
            pltpu.SemaphoreType.DMA,
            pltpu.SemaphoreType.DMA,
            pltpu.SemaphoreType.DMA,
        ],
        compiler_params=_SC_PARAMS,
    )(_scatter_body)
    return fn(msg, dstp_s)


def _msg_kernel(ea_ref, xj_ref, w1_ref, b1_ref, w2_ref, b2_ref, out_ref):
    h = jnp.dot(ea_ref[...], w1_ref[...],
                preferred_element_type=jnp.float32) + b1_ref[...]
    h = 0.5 * h * (1.0 + lax.erf(h * 0.7071067811865476))
    h16 = h.astype(jnp.bfloat16)
    xj = xj_ref[...]
    acc = None
    for i in range(IN_CH):
        gi = jnp.dot(h16, w2_ref[i],
                     preferred_element_type=jnp.float32) + b2_ref[i:i + 1, :]
        term = xj[:, i:i + 1] * gi
        acc = term if acc is None else acc + term
    out_ref[...] = acc


def _tc_message(eap, xj, W1p, b1r, W2r, b2p):
    TE = 2048
    return pl.pallas_call(
        _msg_kernel,
        grid=(EP // TE,),
        in_specs=[
            pl.BlockSpec((TE, 8), lambda i: (i, 0)),
            pl.BlockSpec((TE, 16), lambda i: (i, 0)),
            pl.BlockSpec((8, EMBED), lambda i: (0, 0)),
            pl.BlockSpec((1, EMBED), lambda i: (0, 0)),
            pl.BlockSpec((IN_CH, EMBED, EMBED), lambda i: (0, 0, 0)),
            pl.BlockSpec((8, EMBED), lambda i: (0, 0)),
        ],
        out_specs=pl.BlockSpec((TE, EMBED), lambda i: (i, 0)),
        out_shape=jax.ShapeDtypeStruct((EP, EMBED), jnp.float32),
    )(eap, xj, W1p, b1r, W2r, b2p)


def _final_kernel(acc_ref, cnt_ref, x_ref, root_ref, bias_ref, out_ref):
    summed = acc_ref[0] + acc_ref[1]
    cvec = cnt_ref[0, :, 0:1] + cnt_ref[1, :, 0:1]
    cvec = jnp.maximum(cvec, 1.0)
    xr = jnp.dot(x_ref[...], root_ref[...], preferred_element_type=jnp.float32)
    out_ref[...] = summed / cvec + xr + bias_ref[...]


def _tc_final(accs, cnts, xpad, rootp, biasr):
    TN = 2000
    return pl.pallas_call(
        _final_kernel,
        grid=(N // TN,),
        in_specs=[
            pl.BlockSpec((NC, TN, EMBED), lambda i: (0, i, 0)),
            pl.BlockSpec((NC, TN, 16), lambda i: (0, i, 0)),
            pl.BlockSpec((TN, 16), lambda i: (i, 0)),
            pl.BlockSpec((16, EMBED), lambda i: (0, 0)),
            pl.BlockSpec((1, EMBED), lambda i: (0, 0)),
        ],
        out_specs=pl.BlockSpec((TN, EMBED), lambda i: (i, 0)),
        out_shape=jax.ShapeDtypeStruct((N, EMBED), jnp.float32),
    )(accs, cnts, xpad, rootp, biasr)


def kernel(x, edge_index, edge_attr, W1, b1, W2, b2, root, bias):
    src = edge_index[0].astype(jnp.int32)
    dst = edge_index[1].astype(jnp.int32)
    pad = EP - E
    srcp = jnp.concatenate([src, jnp.zeros((pad,), jnp.int32)]).reshape(NW, CH, K)
    dstp_full = jnp.concatenate([dst, jnp.full((pad,), N, jnp.int32)])
    dstp = dstp_full.reshape(NW, CH, K)
    dstp_s = dstp_full.reshape(NW, CH2, KS)
    eap = jnp.pad(edge_attr, ((0, pad), (0, 2)))
    xpad = jnp.pad(x, ((0, 0), (0, 16 - IN_CH)))
    W1p = jnp.pad(W1, ((0, 2), (0, 0)))
    b1r = b1.reshape(1, EMBED)
    W2r = W2.reshape(EMBED, IN_CH, EMBED).transpose(1, 0, 2).astype(jnp.bfloat16)
    b2p = jnp.pad(b2.reshape(IN_CH, EMBED), ((0, 1), (0, 0)))
    rootp = jnp.pad(root, ((0, 16 - IN_CH), (0, 0)))
    biasr = bias.reshape(1, EMBED)

    xj, cnts = _sc_gather(xpad, srcp, dstp)
    msg = _tc_message(eap, xj, W1p, b1r, W2r, b2p)
    accs = _sc_scatter(msg, dstp_s)
    return _tc_final(accs, cnts, xpad, rootp, biasr)


# speedup vs baseline: 4.5072x; 1.0552x over previous
import functools

import jax
import jax.numpy as jnp
from jax import lax
from jax.experimental import pallas as pl
from jax.experimental.pallas import tpu as pltpu
from jax.experimental.pallas import tpu_sc as plsc

N = 10000
E = 160000
IN_CH = 7
EMBED = 128

NC = 2
NS = 16
NW = NC * NS
K = 128
CH = 40
KS = 64
CH2 = 80
EPW = CH * K
EP = NW * EPW
NA = 10240
RPT = NA // NS
XPT = N // NS
NBUF = 4

_MESH = dict(core_axis_name="c", subcore_axis_name="s", num_cores=NC,
             num_subcores=NS)
_SC_PARAMS = pltpu.CompilerParams(use_tc_tiling_on_sc=False)


def _gather_body(xpad_hbm, srcp_hbm, dstp_hbm, xj_hbm, cnt_hbm,
                 idx_v, didx_v, rows_v, ones_v, zc_v, xs_s, cnt_s, sem):
    c = lax.axis_index("c")
    s = lax.axis_index("s")
    wid = s * NC + c

    pltpu.sync_copy(xpad_hbm.at[pl.ds(s * XPT, XPT)],
                    xs_s.at[pl.ds(s * XPT, XPT)])
    pltpu.sync_copy(srcp_hbm.at[wid], idx_v)
    pltpu.sync_copy(dstp_hbm.at[wid], didx_v)

    zf = jnp.zeros((16,), jnp.float32)
    of = jnp.ones((16,), jnp.float32)

    def fill(r, carry):
        ones_v[r, :] = of
        zc_v[r, :] = zf
        return carry

    lax.fori_loop(0, K, fill, 0)
    for t in range(RPT // K):
        pltpu.sync_copy(zc_v, cnt_s.at[pl.ds(s * RPT + t * K, K)])
    plsc.subcore_barrier()

    def fire(j, carry):
        pltpu.async_copy(xs_s.at[idx_v.at[j]],
                         rows_v.at[pl.ds(j * K, K)], sem)
        return carry

    lax.fori_loop(0, CH, fire, 0)

    def ones_scatter(j, carry):
        pltpu.sync_copy(ones_v, cnt_s.at[didx_v.at[j]], add=True)
        return carry

    lax.fori_loop(0, CH, ones_scatter, 0)

    pltpu.make_async_copy(xj_hbm.at[pl.ds(wid * EPW, EPW)], rows_v,
                          sem).wait()
    pltpu.sync_copy(rows_v, xj_hbm.at[pl.ds(wid * EPW, EPW)])
    plsc.subcore_barrier()
    pltpu.sync_copy(cnt_s.at[pl.ds(s * XPT, XPT)],
                    cnt_hbm.at[c, pl.ds(s * XPT, XPT)])


def _sc_gather(xpad, srcp, dstp):
    fn = functools.partial(
        pl.kernel,
        out_type=(jax.ShapeDtypeStruct((EP, 16), jnp.float32),
                  jax.ShapeDtypeStruct((NC, N, 16), jnp.float32)),
        mesh=plsc.VectorSubcoreMesh(**_MESH),
        scratch_types=[
            pltpu.VMEM((CH, K), jnp.int32),
            pltpu.VMEM((CH, K), jnp.int32),
            pltpu.VMEM((EPW, 16), jnp.float32),
            pltpu.VMEM((K, 16), jnp.float32),
            pltpu.VMEM((K, 16), jnp.float32),
            pltpu.VMEM_SHARED((N, 16), jnp.float32),
            pltpu.VMEM_SHARED((NA, 16), jnp.float32),
            pltpu.SemaphoreType.DMA,
        ],
        compiler_params=_SC_PARAMS,
    )(_gather_body)
    return fn(xpad, srcp, dstp)


def _scatter_body(msg_hbm, dstp_hbm, acc_hbm,
                  idx_v, st0, st1, st2, st3, acc_s, sm0, sm1, sm2, sm3):
    c = lax.axis_index("c")
    s = lax.axis_index("s")
    wid = s * NC + c
    base = wid * EPW
    stages = (st0, st1, st2, st3)
    sems = (sm0, sm1, sm2, sm3)

    zf = jnp.zeros((16,), jnp.float32)

    def zstage(r, carry):
        for cc in range(8):
            st0[r, pl.ds(cc * 16, 16)] = zf
        return carry

    lax.fori_loop(0, KS, zstage, 0)
    for t in range(RPT // KS):
        pltpu.sync_copy(st0, acc_s.at[pl.ds(s * RPT + t * KS, KS)])
    plsc.subcore_barrier()

    pltpu.sync_copy(dstp_hbm.at[wid], idx_v)

    for b in range(NBUF):
        pltpu.async_copy(msg_hbm.at[pl.ds(base + b * KS, KS)], stages[b],
                         sems[b])

    def outer(o, carry):
        for b in range(NBUF):
            j = o * NBUF + b
            pltpu.make_async_copy(msg_hbm.at[pl.ds(base + j * KS, KS)],
                                  stages[b], sems[b]).wait()
            pltpu.sync_copy(stages[b], acc_s.at[idx_v.at[j]], add=True)

            @pl.when(o < CH2 // NBUF - 1)
            def _fire(b=b, j=j):
                pltpu.async_copy(msg_hbm.at[pl.ds(base + (j + NBUF) * KS, KS)],
                                 stages[b], sems[b])

        return carry

    lax.fori_loop(0, CH2 // NBUF, outer, 0)
    plsc.subcore_barrier()

    pltpu.sync_copy(acc_s.at[pl.ds(s * XPT, XPT)],
                    acc_hbm.at[c, pl.ds(s * XPT, XPT)])


def _sc_scatter(msg, dstp_s):
    fn = functools.partial(
        pl.kernel,
        out_type=jax.ShapeDtypeStruct((NC, N, EMBED), jnp.float32),
        mesh=plsc.VectorSubcoreMesh(**_MESH),
        scratch_types=[
            pltpu.VMEM((CH2, KS), jnp.int32),
            pltpu.VMEM((KS, EMBED), jnp.float32),
            pltpu.VMEM((KS, EMBED), jnp.float32),
            pltpu.VMEM((KS, EMBED), jnp.float32),
            pltpu.VMEM((KS, EMBED), jnp.float32),
            pltpu.VMEM_SHARED((NA, EMBED), jnp.float32),
            pltpu.SemaphoreType.DMA,
            pltpu.SemaphoreType.DMA,
            pltpu.SemaphoreType.DMA,
            pltpu.SemaphoreType.DMA,
        ],
        compiler_params=_SC_PARAMS,
    )(_scatter_body)
    return fn(msg, dstp_s)


def _msg_kernel(ea_ref, xj_ref, w1_ref, b1_ref, w2_ref, b2_ref, out_ref):
    h = jnp.dot(ea_ref[...], w1_ref[...],
                preferred_element_type=jnp.float32) + b1_ref[...]
    h = 0.5 * h * (1.0 + lax.erf(h * 0.7071067811865476))
    h16 = h.astype(jnp.bfloat16)
    xj = xj_ref[...]
    xj16 = xj.astype(jnp.bfloat16)
    t = jnp.concatenate([xj16[:, i:i + 1] * h16 for i in range(IN_CH)],
                        axis=1)
    out_ref[...] = (
        jnp.dot(t, w2_ref[...], preferred_element_type=jnp.float32)
        + jnp.dot(xj, b2_ref[...], preferred_element_type=jnp.float32))


def _tc_message(eap, xj, W1p, b1r, W2cat, b2p16):
    TE = 2048
    return pl.pallas_call(
        _msg_kernel,
        grid=(EP // TE,),
        in_specs=[
            pl.BlockSpec((TE, 8), lambda i: (i, 0)),
            pl.BlockSpec((TE, 16), lambda i: (i, 0)),
            pl.BlockSpec((8, EMBED), lambda i: (0, 0)),
            pl.BlockSpec((1, EMBED), lambda i: (0, 0)),
            pl.BlockSpec((IN_CH * EMBED, EMBED), lambda i: (0, 0)),
            pl.BlockSpec((16, EMBED), lambda i: (0, 0)),
        ],
        out_specs=pl.BlockSpec((TE, EMBED), lambda i: (i, 0)),
        out_shape=jax.ShapeDtypeStruct((EP, EMBED), jnp.float32),
    )(eap, xj, W1p, b1r, W2cat, b2p16)


def _final_kernel(acc_ref, cnt_ref, x_ref, root_ref, bias_ref, out_ref):
    summed = acc_ref[0] + acc_ref[1]
    cvec = cnt_ref[0, :, 0:1] + cnt_ref[1, :, 0:1]
    cvec = jnp.maximum(cvec, 1.0)
    xr = jnp.dot(x_ref[...], root_ref[...], preferred_element_type=jnp.float32)
    out_ref[...] = summed / cvec + xr + bias_ref[...]


def _tc_final(accs, cnts, xpad, rootp, biasr):
    TN = 2000
    return pl.pallas_call(
        _final_kernel,
        grid=(N // TN,),
        in_specs=[
            pl.BlockSpec((NC, TN, EMBED), lambda i: (0, i, 0)),
            pl.BlockSpec((NC, TN, 16), lambda i: (0, i, 0)),
            pl.BlockSpec((TN, 16), lambda i: (i, 0)),
            pl.BlockSpec((16, EMBED), lambda i: (0, 0)),
            pl.BlockSpec((1, EMBED), lambda i: (0, 0)),
        ],
        out_specs=pl.BlockSpec((TN, EMBED), lambda i: (i, 0)),
        out_shape=jax.ShapeDtypeStruct((N, EMBED), jnp.float32),
    )(accs, cnts, xpad, rootp, biasr)


def kernel(x, edge_index, edge_attr, W1, b1, W2, b2, root, bias):
    src = edge_index[0].astype(jnp.int32)
    dst = edge_index[1].astype(jnp.int32)
    pad = EP - E
    srcp = jnp.concatenate([src, jnp.zeros((pad,), jnp.int32)]).reshape(NW, CH, K)
    dstp_full = jnp.concatenate([dst, jnp.full((pad,), N, jnp.int32)])
    dstp = dstp_full.reshape(NW, CH, K)
    dstp_s = dstp_full.reshape(NW, CH2, KS)
    eap = jnp.pad(edge_attr, ((0, pad), (0, 2)))
    xpad = jnp.pad(x, ((0, 0), (0, 16 - IN_CH)))
    W1p = jnp.pad(W1, ((0, 2), (0, 0)))
    b1r = b1.reshape(1, EMBED)
    W2cat = (W2.reshape(EMBED, IN_CH, EMBED).transpose(1, 0, 2)
             .reshape(IN_CH * EMBED, EMBED).astype(jnp.bfloat16))
    b2p16 = jnp.pad(b2.reshape(IN_CH, EMBED), ((0, 16 - IN_CH), (0, 0)))
    rootp = jnp.pad(root, ((0, 16 - IN_CH), (0, 0)))
    biasr = bias.reshape(1, EMBED)

    xj, cnts = _sc_gather(xpad, srcp, dstp)
    msg = _tc_message(eap, xj, W1p, b1r, W2cat, b2p16)
    accs = _sc_scatter(msg, dstp_s)
    return _tc_final(accs, cnts, xpad, rootp, biasr)


# speedup vs baseline: 5.5692x; 1.2356x over previous
import functools

import jax
import jax.numpy as jnp
from jax import lax
from jax.experimental import pallas as pl
from jax.experimental.pallas import tpu as pltpu
from jax.experimental.pallas import tpu_sc as plsc

N = 10000
E = 160000
IN_CH = 7
EMBED = 128
EDGE_DIM = 6

NC = 2
NS = 16
NW = NC * NS
K = 128
CH = 40
KS = 64
CH2 = 80
EPW = CH * K
EP = NW * EPW
NA = 10240
RPT = NA // NS
XPT = N // NS
NBUF = 4

_MESH = dict(core_axis_name="c", subcore_axis_name="s", num_cores=NC,
             num_subcores=NS)
_SC_PARAMS = pltpu.CompilerParams(use_tc_tiling_on_sc=False)


def _gather_body(xpad_hbm, srcp_hbm, dstp_hbm, xj_hbm, cnt_hbm,
                 idx_v, didx_v, rows_v, ones_v, zc_v, xs_s, cnt_s, sem):
    c = lax.axis_index("c")
    s = lax.axis_index("s")
    wid = s * NC + c

    pltpu.sync_copy(xpad_hbm.at[pl.ds(s * XPT, XPT)],
                    xs_s.at[pl.ds(s * XPT, XPT)])
    pltpu.sync_copy(srcp_hbm.at[wid], idx_v)
    pltpu.sync_copy(dstp_hbm.at[wid], didx_v)

    zf = jnp.zeros((16,), jnp.float32)
    of = jnp.ones((16,), jnp.float32)

    def fill(r, carry):
        ones_v[r, :] = of
        zc_v[r, :] = zf
        return carry

    lax.fori_loop(0, K, fill, 0)
    for t in range(RPT // K):
        pltpu.sync_copy(zc_v, cnt_s.at[pl.ds(s * RPT + t * K, K)])
    plsc.subcore_barrier()

    def fire(j, carry):
        pltpu.async_copy(xs_s.at[idx_v.at[j]],
                         rows_v.at[pl.ds(j * K, K)], sem)
        return carry

    lax.fori_loop(0, CH, fire, 0)

    def ones_scatter(j, carry):
        pltpu.sync_copy(ones_v, cnt_s.at[didx_v.at[j]], add=True)
        return carry

    lax.fori_loop(0, CH, ones_scatter, 0)

    pltpu.make_async_copy(xj_hbm.at[pl.ds(wid * EPW, EPW)], rows_v,
                          sem).wait()
    pltpu.sync_copy(rows_v, xj_hbm.at[pl.ds(wid * EPW, EPW)])
    plsc.subcore_barrier()
    pltpu.sync_copy(cnt_s.at[pl.ds(s * XPT, XPT)],
                    cnt_hbm.at[c, pl.ds(s * XPT, XPT)])


def _sc_gather(xpad, srcp, dstp):
    fn = functools.partial(
        pl.kernel,
        out_type=(jax.ShapeDtypeStruct((EP, 16), jnp.float32),
                  jax.ShapeDtypeStruct((NC, N, 16), jnp.float32)),
        mesh=plsc.VectorSubcoreMesh(**_MESH),
        scratch_types=[
            pltpu.VMEM((CH, K), jnp.int32),
            pltpu.VMEM((CH, K), jnp.int32),
            pltpu.VMEM((EPW, 16), jnp.float32),
            pltpu.VMEM((K, 16), jnp.float32),
            pltpu.VMEM((K, 16), jnp.float32),
            pltpu.VMEM_SHARED((N, 16), jnp.float32),
            pltpu.VMEM_SHARED((NA, 16), jnp.float32),
            pltpu.SemaphoreType.DMA,
        ],
        compiler_params=_SC_PARAMS,
    )(_gather_body)
    return fn(xpad, srcp, dstp)


def _scatter_body(msg_hbm, dstp_hbm, acc_hbm,
                  idx_v, st0, st1, st2, st3, acc_s, sm0, sm1, sm2, sm3):
    c = lax.axis_index("c")
    s = lax.axis_index("s")
    wid = s * NC + c
    base = wid * EPW
    stages = (st0, st1, st2, st3)
    sems = (sm0, sm1, sm2, sm3)

    zf = jnp.zeros((16,), jnp.float32)

    def zstage(r, carry):
        for cc in range(8):
            st0[r, pl.ds(cc * 16, 16)] = zf
        return carry

    lax.fori_loop(0, KS, zstage, 0)
    for t in range(RPT // KS):
        pltpu.sync_copy(st0, acc_s.at[pl.ds(s * RPT + t * KS, KS)])
    plsc.subcore_barrier()

    pltpu.sync_copy(dstp_hbm.at[wid], idx_v)

    for b in range(NBUF):
        pltpu.async_copy(msg_hbm.at[pl.ds(base + b * KS, KS)], stages[b],
                         sems[b])

    def outer(o, carry):
        for b in range(NBUF):
            j = o * NBUF + b
            pltpu.make_async_copy(msg_hbm.at[pl.ds(base + j * KS, KS)],
                                  stages[b], sems[b]).wait()
            pltpu.sync_copy(stages[b], acc_s.at[idx_v.at[j]], add=True)

            @pl.when(o < CH2 // NBUF - 1)
            def _fire(b=b, j=j):
                pltpu.async_copy(msg_hbm.at[pl.ds(base + (j + NBUF) * KS, KS)],
                                 stages[b], sems[b])

        return carry

    lax.fori_loop(0, CH2 // NBUF, outer, 0)
    plsc.subcore_barrier()

    pltpu.sync_copy(acc_s.at[pl.ds(s * XPT, XPT)],
                    acc_hbm.at[c, pl.ds(s * XPT, XPT)])


def _sc_scatter(msg, dstp_s):
    fn = functools.partial(
        pl.kernel,
        out_type=jax.ShapeDtypeStruct((NC, N, EMBED), jnp.float32),
        mesh=plsc.VectorSubcoreMesh(**_MESH),
        scratch_types=[
            pltpu.VMEM((CH2, KS), jnp.int32),
            pltpu.VMEM((KS, EMBED), jnp.float32),
            pltpu.VMEM((KS, EMBED), jnp.float32),
            pltpu.VMEM((KS, EMBED), jnp.float32),
            pltpu.VMEM((KS, EMBED), jnp.float32),
            pltpu.VMEM_SHARED((NA, EMBED), jnp.float32),
            pltpu.SemaphoreType.DMA,
            pltpu.SemaphoreType.DMA,
            pltpu.SemaphoreType.DMA,
            pltpu.SemaphoreType.DMA,
        ],
        compiler_params=_SC_PARAMS,
    )(_scatter_body)
    return fn(msg, dstp_s)


_TE = 3200


def _msg_kernel(ea_ref, xj_ref, w1_ref, b1_ref, w2_ref, b2_ref, out_ref):
    ea = ea_ref[...]
    hT = lax.dot_general(w1_ref[...], ea, (((0,), (0,)), ((), ())),
                         preferred_element_type=jnp.float32)
    ri = lax.broadcasted_iota(jnp.int32, (EMBED, EMBED), 0)
    ci = lax.broadcasted_iota(jnp.int32, (EMBED, EMBED), 1)
    eye = (ri == ci).astype(jnp.float32)
    h = lax.dot_general(hT, eye, (((0,), (0,)), ((), ())),
                        preferred_element_type=jnp.float32)
    h = h + b1_ref[...]
    h = 0.5 * h * (1.0 + lax.erf(h * 0.7071067811865476))
    h16 = h.astype(jnp.bfloat16)
    xj = xj_ref[...]
    xj16 = xj.astype(jnp.bfloat16)
    t = jnp.concatenate([xj16[:, i:i + 1] * h16 for i in range(IN_CH)],
                        axis=1)
    out_ref[...] = (
        jnp.dot(t, w2_ref[...], preferred_element_type=jnp.float32)
        + jnp.dot(xj, b2_ref[...], preferred_element_type=jnp.float32))


def _tc_message(eaT, xj, W1, b1r, W2cat, b2p16):
    return pl.pallas_call(
        _msg_kernel,
        grid=(E // _TE,),
        in_specs=[
            pl.BlockSpec((EDGE_DIM, _TE), lambda i: (0, i)),
            pl.BlockSpec((_TE, 16), lambda i: (i, 0)),
            pl.BlockSpec((EDGE_DIM, EMBED), lambda i: (0, 0)),
            pl.BlockSpec((1, EMBED), lambda i: (0, 0)),
            pl.BlockSpec((IN_CH * EMBED, EMBED), lambda i: (0, 0)),
            pl.BlockSpec((16, EMBED), lambda i: (0, 0)),
        ],
        out_specs=pl.BlockSpec((_TE, EMBED), lambda i: (i, 0)),
        out_shape=jax.ShapeDtypeStruct((EP, EMBED), jnp.float32),
    )(eaT, xj, W1, b1r, W2cat, b2p16)


def _final_kernel(acc_ref, cnt_ref, x_ref, root_ref, bias_ref, out_ref):
    summed = acc_ref[0] + acc_ref[1]
    cvec = cnt_ref[0, :, 0:1] + cnt_ref[1, :, 0:1]
    cvec = jnp.maximum(cvec, 1.0)
    xr = jnp.dot(x_ref[...], root_ref[...], preferred_element_type=jnp.float32)
    out_ref[...] = summed / cvec + xr + bias_ref[...]


def _tc_final(accs, cnts, xpad, rootp, biasr):
    TN = 2000
    return pl.pallas_call(
        _final_kernel,
        grid=(N // TN,),
        in_specs=[
            pl.BlockSpec((NC, TN, EMBED), lambda i: (0, i, 0)),
            pl.BlockSpec((NC, TN, 16), lambda i: (0, i, 0)),
            pl.BlockSpec((TN, 16), lambda i: (i, 0)),
            pl.BlockSpec((16, EMBED), lambda i: (0, 0)),
            pl.BlockSpec((1, EMBED), lambda i: (0, 0)),
        ],
        out_specs=pl.BlockSpec((TN, EMBED), lambda i: (i, 0)),
        out_shape=jax.ShapeDtypeStruct((N, EMBED), jnp.float32),
    )(accs, cnts, xpad, rootp, biasr)


def kernel(x, edge_index, edge_attr, W1, b1, W2, b2, root, bias):
    src = edge_index[0].astype(jnp.int32)
    dst = edge_index[1].astype(jnp.int32)
    pad = EP - E
    srcp = jnp.concatenate([src, jnp.zeros((pad,), jnp.int32)]).reshape(NW, CH, K)
    dstp_full = jnp.concatenate([dst, jnp.full((pad,), N, jnp.int32)])
    dstp = dstp_full.reshape(NW, CH, K)
    dstp_s = dstp_full.reshape(NW, CH2, KS)
    eaT = edge_attr.T
    xpad = jnp.pad(x, ((0, 0), (0, 16 - IN_CH)))
    b1r = b1.reshape(1, EMBED)
    W2cat = (W2.reshape(EMBED, IN_CH, EMBED).transpose(1, 0, 2)
             .reshape(IN_CH * EMBED, EMBED).astype(jnp.bfloat16))
    b2p16 = jnp.pad(b2.reshape(IN_CH, EMBED), ((0, 16 - IN_CH), (0, 0)))
    rootp = jnp.pad(root, ((0, 16 - IN_CH), (0, 0)))
    biasr = bias.reshape(1, EMBED)

    xj, cnts = _sc_gather(xpad, srcp, dstp)
    msg = _tc_message(eaT, xj, W1, b1r, W2cat, b2p16)
    accs = _sc_scatter(msg, dstp_s)
    return _tc_final(accs, cnts, xpad, rootp, biasr)
